# baseline (speedup 1.0000x reference)
import functools

import jax
import jax.numpy as jnp
from jax import lax
from jax.experimental import pallas as pl
from jax.experimental.pallas import tpu as pltpu
from jax.experimental.pallas import tpu_sc as plsc

B = 4096
S = 200
D = 64

NC = 2
NS = 16
NW = NC * NS

BPW = B // NW
CHUNK = 104
NCHUNK = 2 * BPW
RPB = 2 * CHUNK

_mesh = plsc.VectorSubcoreMesh(
    core_axis_name="c", subcore_axis_name="s", num_cores=NC, num_subcores=NS
)


@functools.partial(
    pl.kernel,
    out_type=jax.ShapeDtypeStruct((B, D), jnp.float32),
    mesh=_mesh,
    compiler_params=pltpu.CompilerParams(use_tc_tiling_on_sc=False),
    scratch_types=[
        pltpu.VMEM((NCHUNK, CHUNK), jnp.int32),
        pltpu.VMEM((2, RPB, D), jnp.float32),
        pltpu.VMEM((BPW, D), jnp.float32),
        pltpu.SemaphoreType.DMA,
        pltpu.SemaphoreType.DMA,
    ],
)
def _pooled_lookup(table_h, idx_h, out_h, idx_v, rows_v, out_v, sem0, sem1):
    sems = (sem0, sem1)
    wid = lax.axis_index("s") * NC + lax.axis_index("c")

    pltpu.sync_copy(idx_h.at[wid], idx_v)

    def issue(b, j):
        pltpu.async_copy(table_h.at[idx_v.at[2 * b]],
                         rows_v.at[j, pl.ds(0, CHUNK)], sems[j])
        pltpu.async_copy(table_h.at[idx_v.at[2 * b + 1]],
                         rows_v.at[j, pl.ds(CHUNK, CHUNK)], sems[j])

    def wait(j):
        pltpu.make_async_copy(table_h.at[pl.ds(0, RPB)], rows_v.at[j],
                              sems[j]).wait()

    def reduce_store(b, j):
        def body(base):
            def step(i, accs):
                r0 = base + 4 * i
                new = []
                for k in range(4):
                    for c in range(4):
                        v = rows_v[j, r0 + k, pl.ds(c * 16, 16)]
                        new.append(accs[4 * k + c] + v)
                return tuple(new)
            return step

        accs = tuple(jnp.zeros((16,), jnp.float32) for _ in range(16))
        accs = lax.fori_loop(0, 25, body(0), accs)
        accs = lax.fori_loop(0, 25, body(CHUNK), accs)
        for c in range(4):
            tot = (accs[c] + accs[4 + c]) + (accs[8 + c] + accs[12 + c])
            out_v[b, pl.ds(c * 16, 16)] = tot * (1.0 / S)

    issue(0, 0)

    def outer(i, carry):
        b0 = 2 * i
        issue(b0 + 1, 1)
        wait(0)
        reduce_store(b0, 0)

        @pl.when(i < BPW // 2 - 1)
        def _():
            issue(b0 + 2, 0)

        wait(1)
        reduce_store(b0 + 1, 1)
        return carry

    lax.fori_loop(0, BPW // 2, outer, 0)
    pltpu.sync_copy(out_v, out_h.at[pl.ds(wid * BPW, BPW)])


def kernel(inputs, table_pri, cvm):
    del cvm
    idx = inputs.reshape(NW, NCHUNK, S // 2)
    idx = jnp.pad(idx, ((0, 0), (0, 0), (0, CHUNK - S // 2)))
    return _pooled_lookup(table_pri, idx)


# speedup vs baseline: 1.9045x; 1.9045x over previous
import functools

import jax
import jax.numpy as jnp
from jax import lax
from jax.experimental import pallas as pl
from jax.experimental.pallas import tpu as pltpu
from jax.experimental.pallas import tpu_sc as plsc

B = 4096
S = 200
D = 64

NC = 2
NS = 16
NW = NC * NS

BPW = B // NW
NB = 4

_mesh = plsc.VectorSubcoreMesh(
    core_axis_name="c", subcore_axis_name="s", num_cores=NC, num_subcores=NS
)


@functools.partial(
    pl.kernel,
    out_type=jax.ShapeDtypeStruct((B, D), jnp.float32),
    mesh=_mesh,
    compiler_params=pltpu.CompilerParams(use_tc_tiling_on_sc=False),
    scratch_types=[
        pltpu.VMEM((S, BPW), jnp.int32),
        pltpu.VMEM((NB, BPW, D), jnp.float32),
        pltpu.VMEM((BPW, D), jnp.float32),
        pltpu.SemaphoreType.DMA,
        pltpu.SemaphoreType.DMA,
        pltpu.SemaphoreType.DMA,
        pltpu.SemaphoreType.DMA,
    ],
)
def _pooled_lookup(table_h, idx_h, out_h, idx_v, acc_v, out_v, s0, s1, s2, s3):
    sems = (s0, s1, s2, s3)
    wid = lax.axis_index("s") * NC + lax.axis_index("c")

    pltpu.sync_copy(idx_h.at[wid], idx_v)

    for k in range(NB):
        pltpu.async_copy(table_h.at[idx_v.at[k]], acc_v.at[k], sems[k])

    def wait(k):
        pltpu.make_async_copy(table_h.at[pl.ds(0, BPW)], acc_v.at[k],
                              sems[k]).wait()

    def round_(i, carry):
        for k in range(NB):
            wait(k)
            pltpu.async_copy(table_h.at[idx_v.at[NB * i + k]], acc_v.at[k],
                             sems[k], add=True)
        return carry

    lax.fori_loop(1, S // NB, round_, 0)
    for k in range(NB):
        wait(k)

    def combine(g, carry):
        for c in range(4):
            sl = pl.ds(c * 16, 16)
            t = (acc_v[0, g, sl] + acc_v[1, g, sl]) + \
                (acc_v[2, g, sl] + acc_v[3, g, sl])
            out_v[g, sl] = t * (1.0 / S)
        return carry

    lax.fori_loop(0, BPW, combine, 0)
    pltpu.sync_copy(out_v, out_h.at[pl.ds(wid * BPW, BPW)])


def kernel(inputs, table_pri, cvm):
    del cvm
    idx = inputs.reshape(NW, BPW, S).transpose(0, 2, 1)
    return _pooled_lookup(table_pri, idx)
